# baseline (device time: 7905 ns/iter reference)
import jax
import jax.numpy as jnp
from jax import lax
from jax.experimental import pallas as pl
from jax.experimental.pallas import tpu as pltpu

N_DEV = 4


def kernel(x):
    m, n = x.shape

    def body(x_hbm, out_hbm, xv, ov, halo_ref, send_sems, recv_sems, dma_sems):
        my = lax.axis_index("i")
        left = (my - 1) % N_DEV
        right = (my + 1) % N_DEV

        cp_in = pltpu.make_async_copy(x_hbm, xv, dma_sems.at[0])
        cp_in.start()

        barrier_sem = pltpu.get_barrier_semaphore()
        for nbr in (left, right):
            pl.semaphore_signal(
                barrier_sem, inc=1,
                device_id=(nbr,), device_id_type=pl.DeviceIdType.MESH,
            )
        pl.semaphore_wait(barrier_sem, 2)
        cp_in.wait()

        rdma_right = pltpu.make_async_remote_copy(
            src_ref=xv.at[pl.ds(m - 1, 1)],
            dst_ref=halo_ref.at[0],
            send_sem=send_sems.at[0],
            recv_sem=recv_sems.at[0],
            device_id=(right,),
            device_id_type=pl.DeviceIdType.MESH,
        )
        rdma_left = pltpu.make_async_remote_copy(
            src_ref=xv.at[pl.ds(0, 1)],
            dst_ref=halo_ref.at[1],
            send_sem=send_sems.at[1],
            recv_sem=recv_sems.at[1],
            device_id=(left,),
            device_id_type=pl.DeviceIdType.MESH,
        )
        rdma_right.start()
        rdma_left.start()

        ov[pl.ds(1, m - 2), :] = (
            0.25 * xv[pl.ds(0, m - 2), :]
            + 0.5 * xv[pl.ds(1, m - 2), :]
            + 0.25 * xv[pl.ds(2, m - 2), :]
        )

        cp_mid = pltpu.make_async_copy(
            ov.at[pl.ds(8, m - 16)], out_hbm.at[pl.ds(8, m - 16)], dma_sems.at[1]
        )
        cp_mid.start()

        rdma_right.wait_recv()
        rdma_left.wait_recv()

        @pl.when(my == 0)
        def _():
            ov[pl.ds(0, 1), :] = xv[pl.ds(0, 1), :]

        @pl.when(my != 0)
        def _():
            ov[pl.ds(0, 1), :] = (
                0.25 * halo_ref[0]
                + 0.5 * xv[pl.ds(0, 1), :]
                + 0.25 * xv[pl.ds(1, 1), :]
            )

        @pl.when(my == N_DEV - 1)
        def _():
            ov[pl.ds(m - 1, 1), :] = xv[pl.ds(m - 1, 1), :]

        @pl.when(my != N_DEV - 1)
        def _():
            ov[pl.ds(m - 1, 1), :] = (
                0.25 * xv[pl.ds(m - 2, 1), :]
                + 0.5 * xv[pl.ds(m - 1, 1), :]
                + 0.25 * halo_ref[1]
            )

        cp_top = pltpu.make_async_copy(
            ov.at[pl.ds(0, 8)], out_hbm.at[pl.ds(0, 8)], dma_sems.at[2]
        )
        cp_bot = pltpu.make_async_copy(
            ov.at[pl.ds(m - 8, 8)], out_hbm.at[pl.ds(m - 8, 8)], dma_sems.at[3]
        )
        cp_top.start()
        cp_bot.start()

        cp_mid.wait()
        cp_top.wait()
        cp_bot.wait()
        rdma_right.wait_send()
        rdma_left.wait_send()

    return pl.pallas_call(
        body,
        out_shape=jax.ShapeDtypeStruct((m, n), x.dtype),
        in_specs=[pl.BlockSpec(memory_space=pl.ANY)],
        out_specs=pl.BlockSpec(memory_space=pl.ANY),
        scratch_shapes=[
            pltpu.VMEM((m, n), x.dtype),
            pltpu.VMEM((m, n), x.dtype),
            pltpu.VMEM((2, 1, n), x.dtype),
            pltpu.SemaphoreType.DMA((2,)),
            pltpu.SemaphoreType.DMA((2,)),
            pltpu.SemaphoreType.DMA((4,)),
        ],
        compiler_params=pltpu.CompilerParams(collective_id=0),
    )(x)
